# baseline (speedup 1.0000x reference)
import jax
import jax.numpy as jnp
from jax.experimental import pallas as pl
from jax.experimental.pallas import tpu as pltpu

DIM = 2048
EPS = 1e-5
IN_ROWS = 2048
OUT_ROWS = 1024


def _ln_add_kernel(idx_ref, x_ref, gamma_ref, beta_ref, emb_ref, o_ref):
    j = pl.program_id(1)
    x = x_ref[pl.ds(j * OUT_ROWS, OUT_ROWS), :]
    mean = jnp.mean(x, axis=-1, keepdims=True)
    xc = x - mean
    var = jnp.mean(xc * xc, axis=-1, keepdims=True)
    inv = jax.lax.rsqrt(var + EPS)
    b = beta_ref[...] + emb_ref[idx_ref[0], :]
    o_ref[...] = xc * inv * gamma_ref[...] + b


def kernel(x, gamma, beta, emb, modality_idx):
    orig_shape = x.shape
    rows = x.size // DIM
    x2 = x.reshape(rows, DIM)
    grid = (rows // IN_ROWS, IN_ROWS // OUT_ROWS)
    idx = jnp.reshape(modality_idx, (1,)).astype(jnp.int32)

    out = pl.pallas_call(
        _ln_add_kernel,
        grid_spec=pltpu.PrefetchScalarGridSpec(
            num_scalar_prefetch=1,
            grid=grid,
            in_specs=[
                pl.BlockSpec((IN_ROWS, DIM), lambda i, j, s: (i, 0)),
                pl.BlockSpec((DIM,), lambda i, j, s: (0,)),
                pl.BlockSpec((DIM,), lambda i, j, s: (0,)),
                pl.BlockSpec(emb.shape, lambda i, j, s: (0, 0)),
            ],
            out_specs=pl.BlockSpec(
                (OUT_ROWS, DIM), lambda i, j, s: (i * 2 + j, 0)
            ),
        ),
        out_shape=jax.ShapeDtypeStruct((rows, DIM), x.dtype),
    )(idx, x2, gamma, beta, emb)
    return out.reshape(orig_shape)
